# baseline (speedup 1.0000x reference)
import functools

import jax
import jax.numpy as jnp
import numpy as np
from jax import lax
from jax.experimental import pallas as pl
from jax.experimental.pallas import tpu as pltpu
from jax.experimental.pallas import tpu_sc as plsc

_REFINE_PORT = 0.1
_R, _LOG_R = 2048, 11
_L, _LOG_L = 128, 7
_CHUNK = 128


def _margin_body(p_ref, m_ref):
    x = p_ref[0]
    a1 = jnp.max(x, axis=0, keepdims=True)
    e = jnp.exp(x - a1)
    s = e[0:1]
    for i in range(1, x.shape[0]):
        s = s + e[i : i + 1]
    sm = e / s
    t1 = jnp.max(sm, axis=0)
    cls = jax.lax.broadcasted_iota(jnp.int32, sm.shape, 0)
    i1 = jnp.min(jnp.where(sm == t1[None, :], cls, sm.shape[0]), axis=0)
    t2 = jnp.max(jnp.where(cls == i1[None, :], -jnp.inf, sm), axis=0)
    m_ref[0, 0] = t1 - t2


def _margin(probs_r):
    B, C, N = probs_r.shape
    NB = 8192
    out = pl.pallas_call(
        _margin_body,
        grid=(B, N // NB),
        in_specs=[pl.BlockSpec((1, C, NB), lambda b, j: (b, 0, j))],
        out_specs=pl.BlockSpec((1, 1, NB), lambda b, j: (b, 0, j)),
        out_shape=jax.ShapeDtypeStruct((B, 1, N), jnp.float32),
    )(probs_r)
    return out[:, 0]


def _roll(x, s, ax):
    n = x.shape[ax]
    s = s % n
    if s == 0:
        return x
    if ax == 0:
        return jnp.concatenate([x[n - s :, :], x[: n - s, :]], axis=0)
    return jnp.concatenate([x[:, n - s :], x[:, : n - s]], axis=1)


def _sort_body(m_ref, ord_ref):
    K = m_ref[0]
    row = jax.lax.broadcasted_iota(jnp.int32, (_R, _L), 0)
    col = jax.lax.broadcasted_iota(jnp.int32, (_R, _L), 1)
    I = row * _L + col

    def ebit(exp):
        if exp < _LOG_R:
            return ((row >> exp) & 1) == 1
        return ((col >> (exp - _LOG_R)) & 1) == 1

    for p in range(1, _LOG_R + _LOG_L + 1):
        for q in range(p - 1, -1, -1):
            high = ebit(q)
            asc = ebit(p)
            if q < _LOG_R:
                sh, ax = 1 << q, 0
            else:
                sh, ax = 1 << (q - _LOG_R), 1
            Km, Kp = _roll(K, -sh, ax), _roll(K, sh, ax)
            Im, Ip = _roll(I, -sh, ax), _roll(I, sh, ax)
            pK = jnp.where(high, Kp, Km)
            pI = jnp.where(high, Ip, Im)
            mb = (K > pK) | ((K == pK) & (I < pI))
            want_mine = mb == (high == asc)
            K = jnp.where(want_mine, K, pK)
            I = jnp.where(want_mine, I, pI)
    ord_ref[0] = jnp.swapaxes(I, 0, 1)


def _sort(margin2):
    B = margin2.shape[0]
    return pl.pallas_call(
        _sort_body,
        grid=(B,),
        in_specs=[pl.BlockSpec((1, _R, _L), lambda b: (b, 0, 0))],
        out_specs=pl.BlockSpec((1, _L, _R), lambda b: (b, 0, 0)),
        out_shape=jax.ShapeDtypeStruct((B, _L, _R), jnp.int32),
    )(margin2)


def _idxmat_body(hr_ref, o_ref, *, C, N):
    b = pl.program_id(0)
    c = pl.program_id(1)
    o_ref[0, 0] = hr_ref[0, 0] + (b * C + c) * N


def _idxmat(hr, C, N):
    B, KP = hr.shape
    return pl.pallas_call(
        functools.partial(_idxmat_body, C=C, N=N),
        grid=(B, C),
        in_specs=[pl.BlockSpec((1, 1, KP), lambda b, c: (b, 0, 0))],
        out_specs=pl.BlockSpec((1, 1, KP), lambda b, c: (b * C + c, 0, 0)),
        out_shape=jax.ShapeDtypeStruct((B * C, 1, KP), jnp.int32),
    )(hr.reshape(B, 1, KP))


def _sc_gather(feats_flat, idxmat3):
    BC, NCHUNK, _ = idxmat3.shape
    info = plsc.get_sparse_core_info()
    NW = info.num_cores * info.num_subcores
    pairs_per_w = BC // NW
    mesh = plsc.VectorSubcoreMesh(core_axis_name="c", subcore_axis_name="s")

    @functools.partial(
        pl.kernel,
        mesh=mesh,
        out_type=jax.ShapeDtypeStruct((BC, NCHUNK, _CHUNK), jnp.float32),
        scratch_types=[
            pltpu.VMEM((NCHUNK, _CHUNK), jnp.int32),
            pltpu.VMEM((NCHUNK, _CHUNK), jnp.float32),
            pltpu.SemaphoreType.DMA,
        ],
    )
    def k(feats_hbm, idx_hbm, out_hbm, idx_v, data_v, sem):
        wid = lax.axis_index("s") * info.num_cores + lax.axis_index("c")
        for t in range(pairs_per_w):
            pair = wid * pairs_per_w + t
            pltpu.sync_copy(idx_hbm.at[pair], idx_v)

            def issue(j, carry):
                pltpu.async_copy(feats_hbm.at[idx_v.at[j]], data_v.at[j], sem)
                return carry

            lax.fori_loop(0, NCHUNK, issue, 0)
            pltpu.make_async_copy(out_hbm.at[pair], data_v, sem).wait()
            pltpu.sync_copy(data_v, out_hbm.at[pair])

    return k(feats_flat, idxmat3)


def kernel(feats, probs):
    B, num_classes, H, W = probs.shape
    C = feats.shape[1]
    N = H * W
    k = int(np.uint64(_REFINE_PORT * N))
    KP = ((k + _CHUNK - 1) // _CHUNK) * _CHUNK
    probs_r = probs.reshape(B, num_classes, N)
    feats_r = feats.reshape(B, C, N)

    margin = _margin(probs_r)
    order = _sort(margin.reshape(B, _R, _L))
    order_flat = order.reshape(B, N)
    hard_region = order_flat[:, :k]

    idxmat = _idxmat(order_flat[:, :KP], C, N)
    hard_feat = _sc_gather(
        feats_r.reshape(B * C * N), idxmat.reshape(B * C, KP // _CHUNK, _CHUNK)
    )
    hard_feat = hard_feat.reshape(B, C, KP)[:, :, :k]
    return (hard_feat, feats_r, hard_region)


# speedup vs baseline: 1.1067x; 1.1067x over previous
import functools

import jax
import jax.numpy as jnp
import numpy as np
from jax import lax
from jax.experimental import pallas as pl
from jax.experimental.pallas import tpu as pltpu
from jax.experimental.pallas import tpu_sc as plsc

_REFINE_PORT = 0.1
_R, _LOG_R = 2048, 11
_L, _LOG_L = 128, 7
_CHUNK = 128


def _margin_body(p_ref, m_ref):
    x = p_ref[0]
    a1 = jnp.max(x, axis=0, keepdims=True)
    e = jnp.exp(x - a1)
    s = e[0:1]
    for i in range(1, x.shape[0]):
        s = s + e[i : i + 1]
    sm = e / s
    t1 = jnp.max(sm, axis=0)
    cls = jax.lax.broadcasted_iota(jnp.int32, sm.shape, 0)
    i1 = jnp.min(jnp.where(sm == t1[None, :], cls, sm.shape[0]), axis=0)
    t2 = jnp.max(jnp.where(cls == i1[None, :], -jnp.inf, sm), axis=0)
    m_ref[0, 0] = t1 - t2


def _margin(probs_r):
    B, C, N = probs_r.shape
    NB = 8192
    out = pl.pallas_call(
        _margin_body,
        grid=(B, N // NB),
        in_specs=[pl.BlockSpec((1, C, NB), lambda b, j: (b, 0, j))],
        out_specs=pl.BlockSpec((1, 1, NB), lambda b, j: (b, 0, j)),
        out_shape=jax.ShapeDtypeStruct((B, 1, N), jnp.float32),
    )(probs_r)
    return out[:, 0]


def _roll(x, s, ax):
    n = x.shape[ax]
    s = s % n
    if s == 0:
        return x
    if ax == 0:
        return jnp.concatenate([x[n - s :, :], x[: n - s, :]], axis=0)
    return jnp.concatenate([x[:, n - s :], x[:, : n - s]], axis=1)


def _sort_body(m_ref, ord_ref):
    K = m_ref[0]
    row = jax.lax.broadcasted_iota(jnp.int32, (_R, _L), 0)
    col = jax.lax.broadcasted_iota(jnp.int32, (_R, _L), 1)
    I = row * _L + col

    def ebit(exp):
        if exp < _LOG_R:
            return ((row >> exp) & 1) == 1
        return ((col >> (exp - _LOG_R)) & 1) == 1

    for p in range(1, _LOG_R + _LOG_L + 1):
        for q in range(p - 1, -1, -1):
            high = ebit(q)
            asc = ebit(p)
            if q < _LOG_R:
                sh, ax = 1 << q, 0
            else:
                sh, ax = 1 << (q - _LOG_R), 1
            Km, Kp = _roll(K, -sh, ax), _roll(K, sh, ax)
            Im, Ip = _roll(I, -sh, ax), _roll(I, sh, ax)
            pK = jnp.where(high, Kp, Km)
            pI = jnp.where(high, Ip, Im)
            mb = (K > pK) | ((K == pK) & (I < pI))
            want_mine = mb == (high == asc)
            K = jnp.where(want_mine, K, pK)
            I = jnp.where(want_mine, I, pI)
    ord_ref[0] = jnp.swapaxes(I, 0, 1)


def _sort(margin2):
    B = margin2.shape[0]
    return pl.pallas_call(
        _sort_body,
        grid=(B,),
        in_specs=[pl.BlockSpec((1, _R, _L), lambda b: (b, 0, 0))],
        out_specs=pl.BlockSpec((1, _L, _R), lambda b: (b, 0, 0)),
        out_shape=jax.ShapeDtypeStruct((B, _L, _R), jnp.int32),
    )(margin2)


def _idxmat_body(hr_ref, o_ref, *, b, C, N):
    c = pl.program_id(0)
    o_ref[0, 0] = hr_ref[0, 0] + (b * C + c) * N


def _idxmat_b(hr, b, C, N):
    _, KP = hr.shape
    return pl.pallas_call(
        functools.partial(_idxmat_body, b=b, C=C, N=N),
        grid=(C,),
        in_specs=[pl.BlockSpec((1, 1, KP), lambda c: (0, 0, 0))],
        out_specs=pl.BlockSpec((1, 1, KP), lambda c: (c, 0, 0)),
        out_shape=jax.ShapeDtypeStruct((C, 1, KP), jnp.int32),
    )(hr.reshape(1, 1, KP))


def _sc_gather(feats_flat, idxmat3):
    BC, NCHUNK, _ = idxmat3.shape
    info = plsc.get_sparse_core_info()
    NW = info.num_cores * info.num_subcores
    pairs_per_w = BC // NW
    mesh = plsc.VectorSubcoreMesh(core_axis_name="c", subcore_axis_name="s")

    @functools.partial(
        pl.kernel,
        mesh=mesh,
        out_type=jax.ShapeDtypeStruct((BC, NCHUNK, _CHUNK), jnp.float32),
        scratch_types=[
            pltpu.VMEM((NCHUNK, _CHUNK), jnp.int32),
            pltpu.VMEM((NCHUNK, _CHUNK), jnp.float32),
            pltpu.SemaphoreType.DMA,
        ],
    )
    def k(feats_hbm, idx_hbm, out_hbm, idx_v, data_v, sem):
        wid = lax.axis_index("s") * info.num_cores + lax.axis_index("c")
        for t in range(pairs_per_w):
            pair = wid * pairs_per_w + t
            pltpu.sync_copy(idx_hbm.at[pair], idx_v)

            def issue(j, carry):
                pltpu.async_copy(feats_hbm.at[idx_v.at[j]], data_v.at[j], sem)
                return carry

            lax.fori_loop(0, NCHUNK, issue, 0)
            pltpu.make_async_copy(out_hbm.at[pair], data_v, sem).wait()
            pltpu.sync_copy(data_v, out_hbm.at[pair])

    return k(feats_flat, idxmat3)


def kernel(feats, probs):
    B, num_classes, H, W = probs.shape
    C = feats.shape[1]
    N = H * W
    k = int(np.uint64(_REFINE_PORT * N))
    KP = ((k + _CHUNK - 1) // _CHUNK) * _CHUNK
    probs_r = probs.reshape(B, num_classes, N)
    feats_r = feats.reshape(B, C, N)

    margin = _margin(probs_r)
    feats_flat = feats_r.reshape(B * C * N)
    orders, feats_hard = [], []
    for b in range(B):
        order_b = _sort(margin[b].reshape(1, _R, _L)).reshape(1, N)
        orders.append(order_b)
        idxmat_b = _idxmat_b(order_b[:, :KP], b, C, N)
        hf_b = _sc_gather(feats_flat, idxmat_b.reshape(C, KP // _CHUNK, _CHUNK))
        feats_hard.append(hf_b.reshape(1, C, KP)[:, :, :k])
    order_flat = jnp.concatenate(orders, axis=0)
    hard_region = order_flat[:, :k]
    hard_feat = jnp.concatenate(feats_hard, axis=0)
    return (hard_feat, feats_r, hard_region)
